# baseline (speedup 1.0000x reference)
import functools

import jax
import jax.numpy as jnp
from jax import lax
from jax.experimental import pallas as pl
from jax.experimental.pallas import tpu as pltpu
from jax.experimental.pallas import tpu_sc as plsc

_UNITS = 1000000
_NEG = 8192
_BATCH = 4096
_DIM = 64

_SB = 2048
_NS = _NEG // _SB
_LOG_UNITS1 = float(jnp.log(jnp.float32(_UNITS + 1.0)))
_LOG_NEG = float(jnp.log(jnp.float32(_NEG)))


def _sc_gather(table, bias, idx):
    ntot = idx.shape[0]
    info = plsc.get_sparse_core_info()
    nw = info.num_cores * info.num_subcores
    bpw = ntot // nw
    assert ntot % nw == 0 and bpw % 8 == 0

    @functools.partial(
        pl.kernel,
        mesh=plsc.VectorSubcoreMesh(core_axis_name="c", subcore_axis_name="s"),
        compiler_params=pltpu.CompilerParams(use_tc_tiling_on_sc=False),
        out_type=(
            jax.ShapeDtypeStruct((ntot, _DIM), jnp.float32),
            jax.ShapeDtypeStruct((ntot,), jnp.float32),
        ),
        scratch_types=[
            pltpu.VMEM((bpw,), jnp.int32),
            pltpu.VMEM((bpw, _DIM), jnp.float32),
            pltpu.VMEM((bpw,), jnp.float32),
            pltpu.SemaphoreType.DMA,
            pltpu.SemaphoreType.DMA,
        ],
    )
    def k(table_hbm, bias_hbm, idx_hbm, rows_out, brows_out,
          idx_v, rows_v, b_v, sem1, sem2):
        wid = lax.axis_index("s") * info.num_cores + lax.axis_index("c")
        base = wid * bpw
        pltpu.sync_copy(idx_hbm.at[pl.ds(base, bpw)], idx_v)
        c1 = pltpu.async_copy(table_hbm.at[idx_v], rows_v, sem1)
        c2 = pltpu.async_copy(bias_hbm.at[idx_v], b_v, sem2)
        c1.wait()
        c2.wait()
        pltpu.sync_copy(rows_v, rows_out.at[pl.ds(base, bpw)])
        pltpu.sync_copy(b_v, brows_out.at[pl.ds(base, bpw)])

    return k(table, bias, idx)


def _neg_log_expected(ids_f32):
    p = (jnp.log(ids_f32 + 2.0) - jnp.log(ids_f32 + 1.0)) / _LOG_UNITS1
    return _LOG_NEG + jnp.log(p)


def _tc_body(tgt_ref, smp_ref, logits_ref, true_w_ref, samp_w_ref,
             true_b_ref, samp_b_ref, out_ref, m_sc, l_sc, tl_sc):
    s = pl.program_id(0)
    logits = logits_ref[...]

    @pl.when(s == 0)
    def _init():
        tw = true_w_ref[...]
        tb = true_b_ref[...]
        tgt_f = tgt_ref[...].astype(jnp.float32)
        tl = (jnp.sum(logits * tw, axis=1, keepdims=True)
              + tb - _neg_log_expected(tgt_f))
        tl_sc[...] = tl
        m_sc[...] = tl
        l_sc[...] = jnp.ones_like(tl)

    w = samp_w_ref[...]
    sb = samp_b_ref[...]
    smp = smp_ref[...]
    adj = sb - _neg_log_expected(smp.astype(jnp.float32))
    x = lax.dot_general(logits, w, (((1,), (1,)), ((), ())),
                        preferred_element_type=jnp.float32)
    x = x + adj
    hit = (tgt_ref[...] == smp).astype(jnp.float32)
    x = x - hit * 1e9

    m_prev = m_sc[...]
    l_prev = l_sc[...]
    m_new = jnp.maximum(m_prev, jnp.max(x, axis=1, keepdims=True))
    l_new = (l_prev * jnp.exp(m_prev - m_new)
             + jnp.sum(jnp.exp(x - m_new), axis=1, keepdims=True))
    m_sc[...] = m_new
    l_sc[...] = l_new

    @pl.when(s == _NS - 1)
    def _fin():
        per_ex = m_sc[...] + jnp.log(l_sc[...]) - tl_sc[...]
        out_ref[...] = jnp.sum(per_ex, axis=0, keepdims=True) / _BATCH


def _tc_loss(tgt_col, smp_row, logits, true_w, samp_w, true_b_col, samp_b_row):
    b = logits.shape[0]
    grid_spec = pltpu.PrefetchScalarGridSpec(
        num_scalar_prefetch=0,
        grid=(_NS,),
        in_specs=[
            pl.BlockSpec((b, 1), lambda s: (0, 0)),
            pl.BlockSpec((1, _SB), lambda s: (0, s)),
            pl.BlockSpec((b, _DIM), lambda s: (0, 0)),
            pl.BlockSpec((b, _DIM), lambda s: (0, 0)),
            pl.BlockSpec((_SB, _DIM), lambda s: (s, 0)),
            pl.BlockSpec((b, 1), lambda s: (0, 0)),
            pl.BlockSpec((1, _SB), lambda s: (0, s)),
        ],
        out_specs=pl.BlockSpec((1, 1), lambda s: (0, 0)),
        scratch_shapes=[
            pltpu.VMEM((b, 1), jnp.float32),
            pltpu.VMEM((b, 1), jnp.float32),
            pltpu.VMEM((b, 1), jnp.float32),
        ],
    )
    loss = pl.pallas_call(
        _tc_body,
        grid_spec=grid_spec,
        out_shape=jax.ShapeDtypeStruct((1, 1), jnp.float32),
        compiler_params=pltpu.CompilerParams(
            dimension_semantics=("arbitrary",),
        ),
    )(tgt_col, smp_row, logits, true_w, samp_w, true_b_col, samp_b_row)
    return loss[0, 0]


def kernel(logits, targets, kernel, bias, sampled):
    idx = jnp.concatenate([targets, sampled])
    rows, brows = _sc_gather(kernel, bias, idx)
    true_w = rows[:_BATCH]
    samp_w = rows[_BATCH:]
    true_b = brows[:_BATCH].reshape(_BATCH, 1)
    samp_b = brows[_BATCH:].reshape(1, _NEG)
    tgt_col = targets.reshape(_BATCH, 1)
    smp_row = sampled.reshape(1, _NEG)
    return _tc_loss(tgt_col, smp_row, logits.reshape(-1, _DIM),
                    true_w, samp_w, true_b, samp_b)


# speedup vs baseline: 1.5687x; 1.5687x over previous
import functools

import jax
import jax.numpy as jnp
from jax import lax
from jax.experimental import pallas as pl
from jax.experimental.pallas import tpu as pltpu
from jax.experimental.pallas import tpu_sc as plsc

_UNITS = 1000000
_NEG = 8192
_BATCH = 4096
_DIM = 64

_SB = 2048
_NS = _NEG // _SB
_LOG_UNITS1 = float(jnp.log(jnp.float32(_UNITS + 1.0)))
_LOG_NEG = float(jnp.log(jnp.float32(_NEG)))


def _sc_gather(table, bias, idx):
    ntot = idx.shape[0]
    info = plsc.get_sparse_core_info()
    nw = info.num_cores * info.num_subcores
    bpw = ntot // nw
    assert ntot % nw == 0 and bpw % 8 == 0

    @functools.partial(
        pl.kernel,
        mesh=plsc.VectorSubcoreMesh(core_axis_name="c", subcore_axis_name="s"),
        out_type=(
            jax.ShapeDtypeStruct((ntot, _DIM), jnp.float32),
            jax.ShapeDtypeStruct((ntot,), jnp.float32),
        ),
        scratch_types=[
            pltpu.VMEM((bpw,), jnp.int32),
            pltpu.VMEM((bpw, _DIM), jnp.float32),
            pltpu.VMEM((bpw,), jnp.float32),
            pltpu.SemaphoreType.DMA,
            pltpu.SemaphoreType.DMA,
        ],
    )
    def k(table_hbm, bias_hbm, idx_hbm, rows_out, brows_out,
          idx_v, rows_v, b_v, sem1, sem2):
        wid = lax.axis_index("s") * info.num_cores + lax.axis_index("c")
        base = wid * bpw
        pltpu.sync_copy(idx_hbm.at[pl.ds(base, bpw)], idx_v)

        cb = pltpu.async_copy(bias_hbm.at[idx_v], b_v, sem2)

        def issue(g, carry):
            vec = idx_v[pl.ds(pl.multiple_of(g * 16, 16), 16)]
            for kk in range(16):
                ij = vec[kk]
                pltpu.async_copy(table_hbm.at[pl.ds(ij, 1), :],
                                 rows_v.at[pl.ds(g * 16 + kk, 1), :], sem1)
            return carry
        lax.fori_loop(0, bpw // 16, issue, 0)

        pltpu.make_async_copy(
            table_hbm.at[pl.ds(0, bpw), :], rows_v, sem1).wait()
        cb.wait()
        pltpu.sync_copy(rows_v, rows_out.at[pl.ds(base, bpw)])
        pltpu.sync_copy(b_v, brows_out.at[pl.ds(base, bpw)])

    return k(table, bias, idx)


def _neg_log_expected(ids_f32):
    p = (jnp.log(ids_f32 + 2.0) - jnp.log(ids_f32 + 1.0)) / _LOG_UNITS1
    return _LOG_NEG + jnp.log(p)


def _tc_body(tgt_ref, smp_ref, logits_ref, true_w_ref, samp_w_ref,
             true_b_ref, samp_b_ref, out_ref, m_sc, l_sc, tl_sc):
    s = pl.program_id(0)
    logits = logits_ref[...]

    @pl.when(s == 0)
    def _init():
        tw = true_w_ref[...]
        tb = true_b_ref[...]
        tgt_f = tgt_ref[...].astype(jnp.float32)
        tl = (jnp.sum(logits * tw, axis=1, keepdims=True)
              + tb - _neg_log_expected(tgt_f))
        tl_sc[...] = tl
        m_sc[...] = tl
        l_sc[...] = jnp.ones_like(tl)

    w = samp_w_ref[...]
    sb = samp_b_ref[...]
    smp = smp_ref[...]
    adj = sb - _neg_log_expected(smp.astype(jnp.float32))
    x = lax.dot_general(logits, w, (((1,), (1,)), ((), ())),
                        preferred_element_type=jnp.float32)
    x = x + adj
    hit = (tgt_ref[...] == smp).astype(jnp.float32)
    x = x - hit * 1e9

    m_prev = m_sc[...]
    l_prev = l_sc[...]
    m_new = jnp.maximum(m_prev, jnp.max(x, axis=1, keepdims=True))
    l_new = (l_prev * jnp.exp(m_prev - m_new)
             + jnp.sum(jnp.exp(x - m_new), axis=1, keepdims=True))
    m_sc[...] = m_new
    l_sc[...] = l_new

    @pl.when(s == _NS - 1)
    def _fin():
        per_ex = m_sc[...] + jnp.log(l_sc[...]) - tl_sc[...]
        out_ref[...] = jnp.sum(per_ex, axis=0, keepdims=True) / _BATCH


def _tc_loss(tgt_col, smp_row, logits, true_w, samp_w, true_b_col, samp_b_row):
    b = logits.shape[0]
    grid_spec = pltpu.PrefetchScalarGridSpec(
        num_scalar_prefetch=0,
        grid=(_NS,),
        in_specs=[
            pl.BlockSpec((b, 1), lambda s: (0, 0)),
            pl.BlockSpec((1, _SB), lambda s: (0, s)),
            pl.BlockSpec((b, _DIM), lambda s: (0, 0)),
            pl.BlockSpec((b, _DIM), lambda s: (0, 0)),
            pl.BlockSpec((_SB, _DIM), lambda s: (s, 0)),
            pl.BlockSpec((b, 1), lambda s: (0, 0)),
            pl.BlockSpec((1, _SB), lambda s: (0, s)),
        ],
        out_specs=pl.BlockSpec((1, 1), lambda s: (0, 0)),
        scratch_shapes=[
            pltpu.VMEM((b, 1), jnp.float32),
            pltpu.VMEM((b, 1), jnp.float32),
            pltpu.VMEM((b, 1), jnp.float32),
        ],
    )
    loss = pl.pallas_call(
        _tc_body,
        grid_spec=grid_spec,
        out_shape=jax.ShapeDtypeStruct((1, 1), jnp.float32),
        compiler_params=pltpu.CompilerParams(
            dimension_semantics=("arbitrary",),
        ),
    )(tgt_col, smp_row, logits, true_w, samp_w, true_b_col, samp_b_row)
    return loss[0, 0]


def kernel(logits, targets, kernel, bias, sampled):
    idx = jnp.concatenate([targets, sampled])
    rows, brows = _sc_gather(kernel, bias, idx)
    true_w = rows[:_BATCH]
    samp_w = rows[_BATCH:]
    true_b = brows[:_BATCH].reshape(_BATCH, 1)
    samp_b = brows[_BATCH:].reshape(1, _NEG)
    tgt_col = targets.reshape(_BATCH, 1)
    smp_row = sampled.reshape(1, _NEG)
    return _tc_loss(tgt_col, smp_row, logits.reshape(-1, _DIM),
                    true_w, samp_w, true_b, samp_b)
